# Initial kernel scaffold; baseline (speedup 1.0000x reference)
import jax
import jax.numpy as jnp
from jax.experimental import pallas as pl


def kernel(x, W1, b1, gamma, beta, W2, b2, W3, b3, W4, b4, emb):
    raise NotImplementedError("write your pallas kernel here")


# speedup vs baseline: 1.7462x; 1.7462x over previous
import functools

import jax
import jax.numpy as jnp
from jax import lax
from jax.experimental import pallas as pl

_OBS = 768
_HID = 1024
_LAT = 256
_VOC = 1024
_HQ = 2
_BATCH = 16384
_COMMIT = 0.5
_BLK = 512


def _fused_kernel(x_ref, w1_ref, b1_ref, gam_ref, bet_ref, w2_ref, b2_ref,
                  w3_ref, b3_ref, w4_ref, b4_ref, emb_ref, embt_ref, out_ref):
    x = x_ref[...]
    h = jnp.dot(x, w1_ref[...], preferred_element_type=jnp.float32) + b1_ref[...]
    mu = jnp.mean(h, axis=1, keepdims=True)
    var = jnp.mean((h - mu) * (h - mu), axis=1, keepdims=True)
    h = (h - mu) / jnp.sqrt(var + 1e-5) * gam_ref[...] + bet_ref[...]
    h = jnp.maximum(h, 0.0)
    latent = jnp.dot(h, w2_ref[...], preferred_element_type=jnp.float32) + b2_ref[...]

    curr = latent
    code_sum = jnp.zeros_like(latent)
    vq_sum = jnp.float32(0.0)
    for l in range(_HQ):
        e = emb_ref[l]
        et = embt_ref[l]
        e2 = jnp.sum(et * et, axis=0, keepdims=True)
        score = e2 - 2.0 * jnp.dot(curr, et, preferred_element_type=jnp.float32)
        m = jnp.min(score, axis=1, keepdims=True)
        iota = lax.broadcasted_iota(jnp.int32, score.shape, 1)
        idx = jnp.min(jnp.where(score <= m, iota, _VOC), axis=1, keepdims=True)
        onehot = (iota == idx).astype(jnp.float32)
        q = jnp.dot(onehot, e, preferred_element_type=jnp.float32)
        diff = q - curr
        vq_sum = vq_sum + jnp.sum(diff * diff)
        code_sum = code_sum + q
        curr = -diff

    h2 = jnp.dot(code_sum, w3_ref[...], preferred_element_type=jnp.float32) + b3_ref[...]
    h2 = jnp.maximum(h2, 0.0)
    recon = jnp.dot(h2, w4_ref[...], preferred_element_type=jnp.float32) + b4_ref[...]
    r = recon - x
    rec_sum = jnp.sum(r * r)

    partial = ((1.0 + _COMMIT) / (_BATCH * _LAT)) * vq_sum \
        + (0.5 / (_BATCH * _OBS)) * rec_sum

    @pl.when(pl.program_id(0) == 0)
    def _init():
        out_ref[...] = jnp.zeros_like(out_ref)

    out_ref[...] += partial


@functools.partial(jax.jit, static_argnames=("interpret",))
def _run(x, W1, b1, gamma, beta, W2, b2, W3, b3, W4, b4, emb, interpret=False):
    embt = jnp.transpose(emb, (0, 2, 1))
    row = lambda v: v.reshape(1, -1)
    grid = _BATCH // _BLK
    full = lambda shape: pl.BlockSpec(shape, lambda i: tuple(0 for _ in shape))
    out = pl.pallas_call(
        _fused_kernel,
        grid=(grid,),
        in_specs=[
            pl.BlockSpec((_BLK, _OBS), lambda i: (i, 0)),
            full((_OBS, _HID)),
            full((1, _HID)),
            full((1, _HID)),
            full((1, _HID)),
            full((_HID, _LAT)),
            full((1, _LAT)),
            full((_LAT, _HID)),
            full((1, _HID)),
            full((_HID, _OBS)),
            full((1, _OBS)),
            full((_HQ, _VOC, _LAT)),
            full((_HQ, _LAT, _VOC)),
        ],
        out_specs=pl.BlockSpec((1, 1), lambda i: (0, 0)),
        out_shape=jax.ShapeDtypeStruct((1, 1), jnp.float32),
        interpret=interpret,
    )(x, W1, row(b1), row(gamma), row(beta), W2, row(b2), W3, row(b3),
      W4, row(b4), emb, embt)
    return out[0, 0]


def kernel(x, W1, b1, gamma, beta, W2, b2, W3, b3, W4, b4, emb):
    return _run(x, W1, b1, gamma, beta, W2, b2, W3, b3, W4, b4, emb)
